# Initial kernel scaffold; baseline (speedup 1.0000x reference)
import jax
import jax.numpy as jnp
from jax.experimental import pallas as pl


def kernel(input, W1, b1, W2, b2, W3, b3, codebook):
    raise NotImplementedError("write your pallas kernel here")


# speedup vs baseline: 1.1190x; 1.1190x over previous
import jax
import jax.numpy as jnp
from jax.experimental import pallas as pl

_B, _CIN, _T = 16, 128, 2048
_H, _D, _K = 384, 256, 1024
_S = 512

_PREC = jax.lax.Precision.DEFAULT


def _shift_r(m):
    z = jnp.zeros((m.shape[0], 1), m.dtype)
    return jnp.concatenate([z, m[:, :-1]], axis=1)


def _shift_l(m):
    z = jnp.zeros((m.shape[0], 1), m.dtype)
    return jnp.concatenate([m[:, 1:], z], axis=1)


def _mm(a, b):
    return jax.lax.dot_general(
        a, b, (((1,), (0,)), ((), ())),
        preferred_element_type=jnp.float32, precision=_PREC)


def _body(x_ref, w1t_ref, w1e_ref, w1o_ref, b1_ref, v0_ref, v12_ref, v3_ref,
          b2_ref, w3_ref, b3_ref, cb_ref, out_ref, loss_ref):
    b = pl.program_id(0)
    x = x_ref[0]
    x0 = x[0:_CIN]
    x3 = x[3 * _CIN:4 * _CIN]
    xA = x[0:3 * _CIN]
    xB = x[_CIN:4 * _CIN]

    h1e = _shift_r(_mm(w1t_ref[0], x3)) + _mm(w1e_ref[...], xA) + b1_ref[...]
    h1o = _mm(w1o_ref[...], xB) + _shift_l(_mm(w1t_ref[1], x0)) + b1_ref[...]
    h1e = jnp.maximum(h1e, 0.0)
    h1o = jnp.maximum(h1o, 0.0)
    h1 = jnp.concatenate([h1e, h1o], axis=0)

    h2 = (_shift_r(_mm(v0_ref[...], h1o)) + _mm(v12_ref[...], h1)
          + _shift_l(_mm(v3_ref[...], h1e)) + b2_ref[...])
    h2 = jnp.maximum(h2, 0.0)

    z = _mm(w3_ref[...], h2) + b3_ref[...]

    cb = cb_ref[...]
    g = _mm(cb, z)
    cnorm = jnp.sum(cb * cb, axis=1, keepdims=True)
    dist = cnorm - 2.0 * g
    m = jnp.min(dist, axis=0, keepdims=True)
    iota = jax.lax.broadcasted_iota(jnp.int32, (_K, _S), 0)
    codes = jnp.min(jnp.where(dist == m, iota, _K), axis=0, keepdims=True)
    onehot = (iota == codes).astype(jnp.float32)
    out_ref[0] = jax.lax.dot_general(
        cb, onehot, (((0,), (0,)), ((), ())),
        preferred_element_type=jnp.float32,
        precision=jax.lax.Precision.HIGHEST)

    znorm = jnp.sum(z * z, axis=0, keepdims=True)
    part = jnp.sum(m + znorm)

    @pl.when(b == 0)
    def _():
        loss_ref[...] = jnp.zeros_like(loss_ref)

    loss_ref[...] += part


def kernel(input, W1, b1, W2, b2, W3, b3, codebook):
    xs = input.reshape(_B, _CIN, _S, 4).transpose(0, 3, 1, 2)
    xs = xs.reshape(_B, 4 * _CIN, _S)
    w1 = W1.transpose(2, 0, 1)
    w1t = jnp.stack([w1[0], w1[3]])
    w1e = jnp.concatenate([w1[1], w1[2], w1[3]], axis=1)
    w1o = jnp.concatenate([w1[0], w1[1], w1[2]], axis=1)
    v = W2.transpose(2, 0, 1)
    v12 = jnp.concatenate([v[1], v[2]], axis=1)
    w3m = W3[:, :, 0]

    out, loss = pl.pallas_call(
        _body,
        grid=(_B,),
        in_specs=[
            pl.BlockSpec((1, 4 * _CIN, _S), lambda b: (b, 0, 0)),
            pl.BlockSpec((2, _H, _CIN), lambda b: (0, 0, 0)),
            pl.BlockSpec((_H, 3 * _CIN), lambda b: (0, 0)),
            pl.BlockSpec((_H, 3 * _CIN), lambda b: (0, 0)),
            pl.BlockSpec((_H, 1), lambda b: (0, 0)),
            pl.BlockSpec((_H, _H), lambda b: (0, 0)),
            pl.BlockSpec((_H, 2 * _H), lambda b: (0, 0)),
            pl.BlockSpec((_H, _H), lambda b: (0, 0)),
            pl.BlockSpec((_H, 1), lambda b: (0, 0)),
            pl.BlockSpec((_D, _H), lambda b: (0, 0)),
            pl.BlockSpec((_D, 1), lambda b: (0, 0)),
            pl.BlockSpec((_K, _D), lambda b: (0, 0)),
        ],
        out_specs=[
            pl.BlockSpec((1, _D, _S), lambda b: (b, 0, 0)),
            pl.BlockSpec((1, 1), lambda b: (0, 0)),
        ],
        out_shape=[
            jax.ShapeDtypeStruct((_B, _D, _S), jnp.float32),
            jax.ShapeDtypeStruct((1, 1), jnp.float32),
        ],
    )(xs, w1t, w1e, w1o, b1[:, None], v[0], v12, v[3], b2[:, None],
      w3m, b3[:, None], codebook)

    loss_s = loss[0, 0] / jnp.float32(_B * _S * _D)
    return out, loss_s, loss_s


# speedup vs baseline: 1.6982x; 1.5176x over previous
import jax
import jax.numpy as jnp
from jax.experimental import pallas as pl

_B, _CIN, _T = 16, 128, 2048
_H, _D, _K = 384, 256, 1024
_S = 512


def _sd(m):
    z = jnp.zeros((1, m.shape[1]), m.dtype)
    return jnp.concatenate([z, m[:-1]], axis=0)


def _su(m):
    z = jnp.zeros((1, m.shape[1]), m.dtype)
    return jnp.concatenate([m[1:], z], axis=0)


def _mm(a, b):
    return jax.lax.dot_general(
        a, b, (((1,), (0,)), ((), ())),
        preferred_element_type=jnp.float32,
        precision=jax.lax.Precision.DEFAULT)


def _mm_tt(a, b):
    return jax.lax.dot_general(
        a, b, (((1,), (1,)), ((), ())),
        preferred_element_type=jnp.float32,
        precision=jax.lax.Precision.DEFAULT)


def _body(x_ref, w12_ref, w0_ref, w3_ref, b1_ref, v12_ref, v0_ref, v3_ref,
          b2_ref, wz_ref, b3_ref, cb_ref, out_ref, loss_ref):
    b = pl.program_id(0)
    xt = x_ref[0].T
    xg = xt.reshape(1024, 256)

    h1 = (_mm(xg, w12_ref[...])
          + _mm(_sd(xg)[:, _CIN:], w0_ref[...])
          + _mm(_su(xg)[:, :_CIN], w3_ref[...])
          + b1_ref[...])
    h1 = jnp.maximum(h1, 0.0)

    hg = h1.reshape(512, 768)
    h2 = (_mm(hg, v12_ref[...])
          + _mm(_sd(hg)[:, _H:], v0_ref[...])
          + _mm(_su(hg)[:, :_H], v3_ref[...])
          + b2_ref[...])
    h2 = jnp.maximum(h2, 0.0)

    z = _mm(h2, wz_ref[...]) + b3_ref[...]

    cb = cb_ref[...]
    g = _mm_tt(cb, z)
    cnorm = jnp.sum(cb * cb, axis=1, keepdims=True)
    dist = cnorm - 2.0 * g
    m = jnp.min(dist, axis=0, keepdims=True)
    iota = jax.lax.broadcasted_iota(jnp.int32, (_K, _S), 0)
    codes = jnp.min(jnp.where(dist == m, iota, _K), axis=0, keepdims=True)
    onehot = (iota == codes).astype(jnp.float32)

    cb_hi = cb.astype(jnp.bfloat16).astype(jnp.float32)
    cb_lo = cb - cb_hi
    out_ref[0] = (jax.lax.dot_general(
        cb_hi, onehot, (((0,), (0,)), ((), ())),
        preferred_element_type=jnp.float32,
        precision=jax.lax.Precision.DEFAULT)
        + jax.lax.dot_general(
        cb_lo, onehot, (((0,), (0,)), ((), ())),
        preferred_element_type=jnp.float32,
        precision=jax.lax.Precision.DEFAULT))

    part = jnp.sum(m) + jnp.sum(z * z)

    @pl.when(b == 0)
    def _():
        loss_ref[...] = jnp.zeros_like(loss_ref)

    loss_ref[...] += part


def kernel(input, W1, b1, W2, b2, W3, b3, codebook):
    w1 = W1.transpose(2, 1, 0)
    w12 = jnp.concatenate([w1[1], w1[2]], axis=0)
    v = W2.transpose(2, 1, 0)
    v12 = jnp.concatenate([v[1], v[2]], axis=0)
    wz = W3[:, :, 0].T

    out, loss = pl.pallas_call(
        _body,
        grid=(_B,),
        in_specs=[
            pl.BlockSpec((1, _CIN, _T), lambda b: (b, 0, 0)),
            pl.BlockSpec((2 * _CIN, _H), lambda b: (0, 0)),
            pl.BlockSpec((_CIN, _H), lambda b: (0, 0)),
            pl.BlockSpec((_CIN, _H), lambda b: (0, 0)),
            pl.BlockSpec((1, _H), lambda b: (0, 0)),
            pl.BlockSpec((2 * _H, _H), lambda b: (0, 0)),
            pl.BlockSpec((_H, _H), lambda b: (0, 0)),
            pl.BlockSpec((_H, _H), lambda b: (0, 0)),
            pl.BlockSpec((1, _H), lambda b: (0, 0)),
            pl.BlockSpec((_H, _D), lambda b: (0, 0)),
            pl.BlockSpec((1, _D), lambda b: (0, 0)),
            pl.BlockSpec((_K, _D), lambda b: (0, 0)),
        ],
        out_specs=[
            pl.BlockSpec((1, _D, _S), lambda b: (b, 0, 0)),
            pl.BlockSpec((1, 1), lambda b: (0, 0)),
        ],
        out_shape=[
            jax.ShapeDtypeStruct((_B, _D, _S), jnp.float32),
            jax.ShapeDtypeStruct((1, 1), jnp.float32),
        ],
    )(input, w12, w1[0], w1[3], b1[None, :], v12, v[0], v[3], b2[None, :],
      wz, b3[None, :], codebook)

    loss_s = loss[0, 0] / jnp.float32(_B * _S * _D)
    return out, loss_s, loss_s


# speedup vs baseline: 1.7263x; 1.0166x over previous
import jax
import jax.numpy as jnp
from jax.experimental import pallas as pl

_B, _CIN, _T = 16, 128, 2048
_H, _D, _K = 384, 256, 1024
_S = 512
_BF = jnp.bfloat16


def _sd(m):
    z = jnp.zeros((1, m.shape[1]), m.dtype)
    return jnp.concatenate([z, m[:-1]], axis=0)


def _su(m):
    z = jnp.zeros((1, m.shape[1]), m.dtype)
    return jnp.concatenate([m[1:], z], axis=0)


def _mm(a, b):
    return jax.lax.dot_general(
        a, b, (((1,), (0,)), ((), ())),
        preferred_element_type=jnp.float32,
        precision=jax.lax.Precision.DEFAULT)


def _mm_tt(a, b):
    return jax.lax.dot_general(
        a, b, (((1,), (1,)), ((), ())),
        preferred_element_type=jnp.float32,
        precision=jax.lax.Precision.DEFAULT)


def _mm_nt(a, b):
    return jax.lax.dot_general(
        a, b, (((0,), (0,)), ((), ())),
        preferred_element_type=jnp.float32,
        precision=jax.lax.Precision.DEFAULT)


def _body(x_ref, w12_ref, w0_ref, w3_ref, b1_ref, v12_ref, v0_ref, v3_ref,
          b2_ref, wz_ref, b3_ref, cbf_ref, cbh_ref, cbl_ref,
          out_ref, loss_ref):
    b = pl.program_id(0)
    xb = x_ref[0].astype(_BF)
    xt = xb.T
    xg = xt.reshape(1024, 256)

    h1 = (_mm(xg, w12_ref[...])
          + _mm(_sd(xg[:, _CIN:]), w0_ref[...])
          + _mm(_su(xg[:, :_CIN]), w3_ref[...])
          + b1_ref[...])
    h1 = jnp.maximum(h1, 0.0).astype(_BF)

    hg = h1.reshape(512, 768)
    h2 = (_mm(hg, v12_ref[...])
          + _mm(_sd(hg[:, _H:]), v0_ref[...])
          + _mm(_su(hg[:, :_H]), v3_ref[...])
          + b2_ref[...])
    h2 = jnp.maximum(h2, 0.0).astype(_BF)

    z = _mm(h2, wz_ref[...]) + b3_ref[...]

    cbf = cbf_ref[...]
    g = _mm_tt(cbh_ref[...], z.astype(_BF))
    cnorm = jnp.sum(cbf * cbf, axis=1, keepdims=True)
    dist = cnorm - 2.0 * g
    m = jnp.min(dist, axis=0, keepdims=True)
    iota = jax.lax.broadcasted_iota(jnp.int32, (_K, _S), 0)
    codes = jnp.min(jnp.where(dist == m, iota, _K), axis=0, keepdims=True)
    onehot = (iota == codes).astype(_BF)

    out_ref[0] = _mm_nt(cbh_ref[...], onehot) + _mm_nt(cbl_ref[...], onehot)

    part = jnp.sum(m) + jnp.sum(z * z)

    @pl.when(b == 0)
    def _():
        loss_ref[...] = jnp.zeros_like(loss_ref)

    loss_ref[...] += part


def kernel(input, W1, b1, W2, b2, W3, b3, codebook):
    w1 = W1.transpose(2, 1, 0).astype(_BF)
    w12 = jnp.concatenate([w1[1], w1[2]], axis=0)
    v = W2.transpose(2, 1, 0).astype(_BF)
    v12 = jnp.concatenate([v[1], v[2]], axis=0)
    wz = W3[:, :, 0].T.astype(_BF)
    cb_hi = codebook.astype(_BF)
    cb_lo = (codebook - cb_hi.astype(jnp.float32)).astype(_BF)

    out, loss = pl.pallas_call(
        _body,
        grid=(_B,),
        in_specs=[
            pl.BlockSpec((1, _CIN, _T), lambda b: (b, 0, 0)),
            pl.BlockSpec((2 * _CIN, _H), lambda b: (0, 0)),
            pl.BlockSpec((_CIN, _H), lambda b: (0, 0)),
            pl.BlockSpec((_CIN, _H), lambda b: (0, 0)),
            pl.BlockSpec((1, _H), lambda b: (0, 0)),
            pl.BlockSpec((2 * _H, _H), lambda b: (0, 0)),
            pl.BlockSpec((_H, _H), lambda b: (0, 0)),
            pl.BlockSpec((_H, _H), lambda b: (0, 0)),
            pl.BlockSpec((1, _H), lambda b: (0, 0)),
            pl.BlockSpec((_H, _D), lambda b: (0, 0)),
            pl.BlockSpec((1, _D), lambda b: (0, 0)),
            pl.BlockSpec((_K, _D), lambda b: (0, 0)),
            pl.BlockSpec((_K, _D), lambda b: (0, 0)),
            pl.BlockSpec((_K, _D), lambda b: (0, 0)),
        ],
        out_specs=[
            pl.BlockSpec((1, _D, _S), lambda b: (b, 0, 0)),
            pl.BlockSpec((1, 1), lambda b: (0, 0)),
        ],
        out_shape=[
            jax.ShapeDtypeStruct((_B, _D, _S), jnp.float32),
            jax.ShapeDtypeStruct((1, 1), jnp.float32),
        ],
    )(input, w12, w1[0], w1[3], b1[None, :], v12, v[0], v[3], b2[None, :],
      wz, b3[None, :], codebook, cb_hi, cb_lo)

    loss_s = loss[0, 0] / jnp.float32(_B * _S * _D)
    return out, loss_s, loss_s
